# baseline (speedup 1.0000x reference)
import functools

import jax
import jax.numpy as jnp
from jax import lax
from jax.experimental import pallas as pl
from jax.experimental.pallas import tpu as pltpu
from jax.experimental.pallas import tpu_sc as plsc

B = 16384
D = 64
NW = 32
BPW = B // NW
C = 64
NCHUNK = BPW // C
L = 16
NG = C // L
N_REL = 500
N_TIME = 366

_mesh = plsc.VectorSubcoreMesh(core_axis_name="c", subcore_axis_name="s")


@functools.partial(
    pl.kernel,
    mesh=_mesh,
    compiler_params=pltpu.CompilerParams(needs_layout_passes=False),
    out_type=(
        jax.ShapeDtypeStruct((B,), jnp.float32),
        jax.ShapeDtypeStruct((B,), jnp.float32),
    ),
    scratch_types=[
        pltpu.VMEM((BPW,), jnp.int32),
        pltpu.VMEM((BPW,), jnp.int32),
        pltpu.VMEM((BPW,), jnp.int32),
        pltpu.VMEM((BPW,), jnp.int32),
        pltpu.VMEM((BPW,), jnp.int32),
        pltpu.VMEM((BPW,), jnp.int32),
        pltpu.VMEM((BPW,), jnp.int32),
        pltpu.VMEM((C,), jnp.int32),
        pltpu.VMEM((C,), jnp.int32),
        pltpu.VMEM((C,), jnp.int32),
        pltpu.VMEM((C,), jnp.int32),
        pltpu.VMEM((C, 2 * D), jnp.float32),
        pltpu.VMEM((C, 2 * D), jnp.float32),
        pltpu.VMEM((C, 2 * D), jnp.float32),
        pltpu.VMEM((C, 2 * D), jnp.float32),
        pltpu.VMEM((N_REL * D,), jnp.float32),
        pltpu.VMEM((N_TIME * D,), jnp.float32),
        pltpu.VMEM((BPW,), jnp.float32),
        pltpu.VMEM((BPW,), jnp.float32),
        pltpu.SemaphoreType.DMA,
    ],
)
def _score_kernel(h_hbm, t_hbm, nh_hbm, nt_hbm, r_hbm, st_hbm, et_hbm,
                  ent_hbm, rel_hbm, time_hbm, pos_hbm, neg_hbm,
                  hi_v, ti_v, nhi_v, nti_v, ri_v, si_v, ei_v,
                  hp_v, tp_v, nhp_v, ntp_v,
                  h_v, t_v, nh_v, nt_v, rel_c, time_c,
                  pos_v, neg_v, sem):
    wid = lax.axis_index("s") * 2 + lax.axis_index("c")
    wb = wid * BPW
    pltpu.sync_copy(h_hbm.at[pl.ds(wb, BPW)], hi_v)
    pltpu.sync_copy(t_hbm.at[pl.ds(wb, BPW)], ti_v)
    pltpu.sync_copy(nh_hbm.at[pl.ds(wb, BPW)], nhi_v)
    pltpu.sync_copy(nt_hbm.at[pl.ds(wb, BPW)], nti_v)
    pltpu.sync_copy(r_hbm.at[pl.ds(wb, BPW)], ri_v)
    pltpu.sync_copy(st_hbm.at[pl.ds(wb, BPW)], si_v)
    pltpu.sync_copy(et_hbm.at[pl.ds(wb, BPW)], ei_v)
    pltpu.sync_copy(rel_hbm, rel_c)
    pltpu.sync_copy(time_hbm, time_c)

    def chunk_body(c, _):
        base = c * C

        @plsc.parallel_loop(0, C, step=L)
        def mk_pair(k):
            sl = pl.ds(base + k, L)
            hp_v[pl.ds(k, L)] = lax.shift_right_logical(hi_v[sl], 1)
            tp_v[pl.ds(k, L)] = lax.shift_right_logical(ti_v[sl], 1)
            nhp_v[pl.ds(k, L)] = lax.shift_right_logical(nhi_v[sl], 1)
            ntp_v[pl.ds(k, L)] = lax.shift_right_logical(nti_v[sl], 1)

        cps = [
            pltpu.async_copy(ent_hbm.at[hp_v], h_v, sem),
            pltpu.async_copy(ent_hbm.at[tp_v], t_v, sem),
            pltpu.async_copy(ent_hbm.at[nhp_v], nh_v, sem),
            pltpu.async_copy(ent_hbm.at[ntp_v], nt_v, sem),
        ]
        for cp in cps:
            cp.wait()

        @plsc.parallel_loop(0, NG)
        def grp(g):
            sl = pl.ds(base + g * L, L)
            slot = lax.iota(jnp.int32, L) + g * L
            hcol = (hi_v[sl] & 1) * D
            tcol = (ti_v[sl] & 1) * D
            nhcol = (nhi_v[sl] & 1) * D
            ntcol = (nti_v[sl] & 1) * D
            rbase = ri_v[sl] * D
            sbase = si_v[sl] * D
            ebase = ei_v[sl] * D
            accp = jnp.zeros((L,), jnp.float32)
            accn = jnp.zeros((L,), jnp.float32)
            for d in range(D):
                hv = plsc.load_gather(h_v, [slot, hcol + d])
                tv = plsc.load_gather(t_v, [slot, tcol + d])
                nhv = plsc.load_gather(nh_v, [slot, nhcol + d])
                ntv = plsc.load_gather(nt_v, [slot, ntcol + d])
                rv = plsc.load_gather(rel_c, [rbase + d])
                tsv = plsc.load_gather(time_c, [sbase + d])
                tev = plsc.load_gather(time_c, [ebase + d])
                trans = rv + 0.5 * (tsv + tev)
                accp = accp + jnp.abs(hv + trans - tv)
                accn = accn + jnp.abs(nhv + trans - ntv)
            pos_v[sl] = -accp
            neg_v[sl] = -accn

        return 0

    lax.fori_loop(0, NCHUNK, chunk_body, 0)

    pltpu.sync_copy(pos_v, pos_hbm.at[pl.ds(wb, BPW)])
    pltpu.sync_copy(neg_v, neg_hbm.at[pl.ds(wb, BPW)])


def kernel(heads, tails, relations, start_time, end_time,
           negative_heads, negative_tails, ent_emb, rel_emb, time_emb):
    pos, neg = _score_kernel(
        heads.astype(jnp.int32), tails.astype(jnp.int32),
        negative_heads.astype(jnp.int32), negative_tails.astype(jnp.int32),
        relations.astype(jnp.int32), start_time.astype(jnp.int32),
        end_time.astype(jnp.int32),
        ent_emb.reshape(500000, 2 * D),
        rel_emb.reshape(-1), time_emb.reshape(-1))
    return pos, neg
